# baseline (speedup 1.0000x reference)
import functools

import jax
import jax.numpy as jnp
from jax import lax
from jax.experimental import pallas as pl
from jax.experimental.pallas import tpu as pltpu
from jax.experimental.pallas import tpu_sc as plsc

BATCH = 4096
ENT = 100
DIM = 64
NC, NS = 2, 16
NW = NC * NS
BPW = BATCH // NW
IPW = BPW * ENT
OPW = 2 * IPW
NCHUNK = IPW // 16
GB = 2
GR = GB * 2 * ENT
NG = OPW // GR
EPAD = 112


def _sc_body(x_hbm, tab_hbm, out_hbm, xv, comb, mval, gbuf0, gbuf1,
             gs0, gs1, os0, os1):
    wid = lax.axis_index("s") * NC + lax.axis_index("c")

    pltpu.sync_copy(x_hbm.at[pl.ds(wid * IPW, IPW)], xv)

    def ibody(i, carry):
        base = i * 16
        v = xv[pl.ds(base, 16)]
        pos = base + lax.iota(jnp.int32, 16)
        neg = v < 0
        cidx = jnp.where(neg, 0, v)
        b = pos // ENT
        e = pos - b * ENT
        dv = b * (2 * ENT) + e
        plsc.store_scatter(comb, [dv], cidx)
        plsc.store_scatter(mval, [b * EPAD + e],
                           jnp.where(neg, 0.0, 1.0).astype(jnp.float32))
        return carry

    lax.fori_loop(0, NCHUNK, ibody, 0)

    out_base = wid * BPW
    bufs = ((gbuf0, gs0, os0), (gbuf1, gs1, os1))

    def gbody(t2, carry):
        for bi in range(2):
            buf, gsem, osem = bufs[bi]
            t = t2 * 2 + bi

            @pl.when(t2 > 0)
            def _wait_prev_out():
                pltpu.make_async_copy(buf, out_hbm.at[pl.ds(0, GB)],
                                      osem).wait()

            descs = []
            for c in range(GB):
                descs.append(pltpu.async_copy(
                    tab_hbm.at[comb.at[pl.ds(t * GR + c * 2 * ENT, ENT)]],
                    buf.at[c, 0], gsem))
            for c in range(GB):
                mbase = (t * GB + c) * EPAD
                for j in range(EPAD // 16):
                    m16 = mval[pl.ds(mbase + j * 16, 16)]
                    for l in range(16):
                        e = j * 16 + l
                        if e >= ENT:
                            continue
                        splat = jnp.full((16,), m16[l], jnp.float32)
                        for k in range(DIM // 16):
                            buf[c, 1, e, pl.ds(k * 16, 16)] = splat
            for d in descs:
                d.wait()
            pltpu.async_copy(buf, out_hbm.at[pl.ds(out_base + t * GB, GB)],
                             osem)
        return carry

    lax.fori_loop(0, NG // 2, gbody, 0)

    for buf, _, osem in bufs:
        pltpu.make_async_copy(buf, out_hbm.at[pl.ds(0, GB)], osem).wait()


_sc_call = functools.partial(
    pl.kernel,
    out_type=jax.ShapeDtypeStruct((BATCH, 2, ENT, DIM), jnp.float32),
    mesh=plsc.VectorSubcoreMesh(core_axis_name="c", subcore_axis_name="s",
                                num_cores=NC, num_subcores=NS),
    scratch_types=[
        pltpu.VMEM((IPW,), jnp.int32),
        pltpu.VMEM((OPW,), jnp.int32),
        pltpu.VMEM((BPW * EPAD,), jnp.float32),
        pltpu.VMEM((GB, 2, ENT, DIM), jnp.float32),
        pltpu.VMEM((GB, 2, ENT, DIM), jnp.float32),
        pltpu.SemaphoreType.DMA,
        pltpu.SemaphoreType.DMA,
        pltpu.SemaphoreType.DMA,
        pltpu.SemaphoreType.DMA,
    ],
    compiler_params=pltpu.CompilerParams(use_tc_tiling_on_sc=False,
                                         needs_layout_passes=False),
)(_sc_body)


def kernel(x, entity_vectors):
    return _sc_call(x.reshape(-1), entity_vectors)


# speedup vs baseline: 7.9587x; 7.9587x over previous
import functools

import jax
import jax.numpy as jnp
from jax import lax
from jax.experimental import pallas as pl
from jax.experimental.pallas import tpu as pltpu
from jax.experimental.pallas import tpu_sc as plsc

BATCH = 4096
ENT = 100
DIM = 64
NC, NS = 2, 16
NW = NC * NS
NVW = NW // 2
CB = 256
NCK = BATCH // CB
DH, DL, BH, BL = DIM // 8, 8, BATCH // 128, 128


def _sc_body(xt_hbm, tab_hbm, out_hbm, xrow, cidx, gb0, gb1, pb0, pb1, slab,
             gs0, gs1, os0, os1, msem):
    wid = lax.axis_index("s") * NC + lax.axis_index("c")

    vk = jnp.where(wid < NVW, wid, wid - NVW)
    nplanes = jnp.where(vk < ENT % NVW, ENT // NVW + 1, ENT // NVW)

    gbufs = ((gb0, gs0, os0), (gb1, gs1, os1))

    def fire_gathers(k, buf, gsem):
        descs = []
        for j in range(CB // 128):
            descs.append(pltpu.async_copy(
                tab_hbm.at[cidx.at[pl.ds(k * CB + j * 128, 128)]],
                buf.at[pl.ds(j * 128, 128)], gsem))
        return descs

    @pl.when(wid < NVW)
    def _vec_planes():
        def plane(i, carry):
            e = vk + NVW * i
            pltpu.sync_copy(xt_hbm.at[e], xrow)

            def clean(j, c2):
                v = xrow[pl.ds(j * 16, 16)]
                cidx[pl.ds(j * 16, 16)] = jnp.where(v < 0, 0, v)
                return c2

            lax.fori_loop(0, BATCH // 16, clean, 0)

            fire_gathers(0, gb0, gs0)

            def chunk2(k2, c2):
                for bi in range(2):
                    k = k2 * 2 + bi
                    buf, gsem, osem = gbufs[bi]
                    pbuf = pb0 if bi == 0 else pb1

                    @pl.when(k2 * 2 + bi < NCK - 1)
                    def _next():
                        nbuf, ngsem, _ = gbufs[1 - bi]
                        fire_gathers(k + 1, nbuf, ngsem)

                    for j in range(CB // 128):
                        pltpu.make_async_copy(
                            tab_hbm.at[cidx.at[pl.ds(0, 128)]],
                            buf.at[pl.ds(j * 128, 128)], gsem).wait()

                    @pl.when(c2 + bi >= 2)
                    def _drain():
                        pltpu.make_async_copy(
                            pbuf, out_hbm.at[0, 0, :, pl.ds(0, CB // 128)],
                            osem).wait()

                    def dhl(dh, c3):
                        for dl in range(DL):
                            dvec = jnp.full((16,), dh * 8 + dl, jnp.int32)
                            for bh in range(CB // 128):
                                for blg in range(8):
                                    rows = (bh * 128 + blg * 16
                                            + lax.iota(jnp.int32, 16))
                                    v = plsc.load_gather(buf, [rows, dvec])
                                    pbuf[dh, bh, dl,
                                         pl.ds(blg * 16, 16)] = v
                        return c3

                    lax.fori_loop(0, DH, dhl, 0)
                    pltpu.async_copy(
                        pbuf,
                        out_hbm.at[0, e, :, pl.ds(k * (CB // 128),
                                                  CB // 128)],
                        osem)
                return c2 + 2

            return lax.fori_loop(0, NCK // 2, chunk2, carry)

        total = lax.fori_loop(0, nplanes, plane, 0)

        @pl.when(total > 0)
        def _final_drain():
            for pbuf, osem in ((pb0, os0), (pb1, os1)):
                pltpu.make_async_copy(
                    pbuf, out_hbm.at[0, 0, :, pl.ds(0, CB // 128)],
                    osem).wait()

    @pl.when(wid >= NVW)
    def _mask_planes():
        def plane(i, carry):
            e = vk + NVW * i

            @pl.when(i > 0)
            def _drain_prev():
                for dh in range(DH):
                    pltpu.make_async_copy(slab, out_hbm.at[1, 0, 0],
                                          msem).wait()

            pltpu.sync_copy(xt_hbm.at[e], xrow)

            def bh_body(bh, c2):
                for blg in range(8):
                    v = xrow[pl.ds(bh * 128 + blg * 16, 16)]
                    m = jnp.where(v < 0, 0.0, 1.0).astype(jnp.float32)
                    for dl in range(DL):
                        slab[bh, dl, pl.ds(blg * 16, 16)] = m
                return c2

            lax.fori_loop(0, BH, bh_body, 0)
            for dh in range(DH):
                pltpu.async_copy(slab, out_hbm.at[1, e, dh], msem)
            return carry + 1

        total = lax.fori_loop(0, nplanes, plane, 0)

        @pl.when(total > 0)
        def _final_drain():
            for dh in range(DH):
                pltpu.make_async_copy(slab, out_hbm.at[1, 0, 0], msem).wait()


_sc_call = functools.partial(
    pl.kernel,
    out_type=jax.ShapeDtypeStruct((2, ENT, DH, BH, DL, BL), jnp.float32),
    mesh=plsc.VectorSubcoreMesh(core_axis_name="c", subcore_axis_name="s",
                                num_cores=NC, num_subcores=NS),
    scratch_types=[
        pltpu.VMEM((BATCH,), jnp.int32),
        pltpu.VMEM((BATCH,), jnp.int32),
        pltpu.VMEM((CB, DIM), jnp.float32),
        pltpu.VMEM((CB, DIM), jnp.float32),
        pltpu.VMEM((DH, CB // 128, DL, BL), jnp.float32),
        pltpu.VMEM((DH, CB // 128, DL, BL), jnp.float32),
        pltpu.VMEM((BH, DL, BL), jnp.float32),
        pltpu.SemaphoreType.DMA,
        pltpu.SemaphoreType.DMA,
        pltpu.SemaphoreType.DMA,
        pltpu.SemaphoreType.DMA,
        pltpu.SemaphoreType.DMA,
    ],
    compiler_params=pltpu.CompilerParams(use_tc_tiling_on_sc=False,
                                         needs_layout_passes=False),
)(_sc_body)


def kernel(x, entity_vectors):
    out6 = _sc_call(x.T, entity_vectors)
    return jnp.transpose(out6, (3, 5, 0, 1, 2, 4)).reshape(BATCH, 2, ENT, DIM)
